# baseline (speedup 1.0000x reference)
import jax
import jax.numpy as jnp
from jax.experimental import pallas as pl
from jax.experimental.pallas import tpu as pltpu

_B, _C, _H, _W = 16, 3, 384, 384
_P = 16
_PATCH_DIM = _C * _P * _P
_HID = 768
_FF = 1536
_D = 256
_K = 8192
_N_TOK = (_H // _P) * (_W // _P)
_T = _B * _N_TOK
_BT = 512
_NB = _T // _BT
_CK = 2048


def _fused_kernel(p_ref, we_ref, be_ref, w1_ref, b1_ref, w2_ref, b2_ref,
                  cb_ref, m_ref, out_ref, c2_ref, hnegt_ref, h2_ref):
    @pl.when(pl.program_id(0) == 0)
    def _():
        cb = cb_ref[...]
        c2_ref[...] = jnp.sum(cb * cb, axis=1, keepdims=True)

    hnegt = hnegt_ref[...]
    h2 = h2_ref[...]
    run_min = jnp.full((1, _BT), jnp.inf, dtype=jnp.float32)
    run_idx = jnp.zeros((1, _BT), dtype=jnp.int32)
    for c in range(_K // _CK):
        a = jnp.dot(cb_ref[c * _CK:(c + 1) * _CK, :], hnegt,
                    preferred_element_type=jnp.float32)
        d2 = (h2 + a) + c2_ref[c * _CK:(c + 1) * _CK, :]
        cmin = jnp.min(d2, axis=0, keepdims=True)
        cidx = jnp.argmin(d2, axis=0).astype(jnp.int32).reshape(1, _BT) + c * _CK
        better = cmin < run_min
        run_idx = jnp.where(better, cidx, run_idx)
        run_min = jnp.where(better, cmin, run_min)
    idx = run_idx.reshape(_BT)
    out_ref[0, 0, :] = jnp.where(m_ref[0, 0, :] != 0, idx, -1)

    h = jnp.dot(p_ref[...], we_ref[...], preferred_element_type=jnp.float32) + be_ref[...]
    h = jax.nn.gelu(h)
    h = jnp.dot(h, w1_ref[...], preferred_element_type=jnp.float32) + b1_ref[...]
    h = jax.nn.gelu(h)
    h = jnp.dot(h, w2_ref[...], preferred_element_type=jnp.float32) + b2_ref[...]
    h2_ref[...] = jnp.sum(h * h, axis=1, keepdims=True).reshape(1, _BT)
    hnegt_ref[...] = (-2.0 * h).T


def kernel(x, mask, W_e, b_e, W1, b1, W2, b2, codebook):
    xp = x.reshape(_B, _C, _H // _P, _P, _W // _P, _P)
    xp = xp.transpose(0, 2, 4, 1, 3, 5).reshape(_T, _PATCH_DIM)
    m = mask.reshape(_NB, 1, _BT).astype(jnp.int32)
    out = pl.pallas_call(
        _fused_kernel,
        grid=(_NB + 1,),
        in_specs=[
            pl.BlockSpec((_BT, _PATCH_DIM),
                         lambda i: (jnp.minimum(i, _NB - 1), 0)),
            pl.BlockSpec((_PATCH_DIM, _HID), lambda i: (0, 0)),
            pl.BlockSpec((1, _HID), lambda i: (0, 0)),
            pl.BlockSpec((_HID, _FF), lambda i: (0, 0)),
            pl.BlockSpec((1, _FF), lambda i: (0, 0)),
            pl.BlockSpec((_FF, _D), lambda i: (0, 0)),
            pl.BlockSpec((1, _D), lambda i: (0, 0)),
            pl.BlockSpec((_K, _D), lambda i: (0, 0)),
            pl.BlockSpec((1, 1, _BT), lambda i: (jnp.maximum(i - 1, 0), 0, 0)),
        ],
        out_specs=pl.BlockSpec((1, 1, _BT), lambda i: (jnp.maximum(i - 1, 0), 0, 0)),
        out_shape=jax.ShapeDtypeStruct((_NB, 1, _BT), jnp.int32),
        scratch_shapes=[
            pltpu.VMEM((_K, 1), jnp.float32),
            pltpu.VMEM((_D, _BT), jnp.float32),
            pltpu.VMEM((1, _BT), jnp.float32),
        ],
    )(xp, W_e, b_e.reshape(1, _HID), W1, b1.reshape(1, _FF), W2,
      b2.reshape(1, _D), codebook, m)
    return out.reshape(_B, _N_TOK)


# speedup vs baseline: 1.0125x; 1.0125x over previous
import jax
import jax.numpy as jnp
from jax.experimental import pallas as pl
from jax.experimental.pallas import tpu as pltpu

_B, _C, _H, _W = 16, 3, 384, 384
_P = 16
_PATCH_DIM = _C * _P * _P
_HID = 768
_FF = 1536
_D = 256
_K = 8192
_N_TOK = (_H // _P) * (_W // _P)
_T = _B * _N_TOK
_BT = 256
_NB = _T // _BT
_CK = 2048


def _fused_kernel(p_ref, we_ref, be_ref, w1_ref, b1_ref, w2_ref, b2_ref,
                  cb_ref, m_ref, out_ref, c2_ref, hnegt_ref, h2_ref):
    @pl.when(pl.program_id(0) == 0)
    def _():
        cb = cb_ref[...]
        c2_ref[...] = jnp.sum(cb * cb, axis=1, keepdims=True)

    hnegt = hnegt_ref[...]
    h2 = h2_ref[...]
    run_min = jnp.full((1, _BT), jnp.inf, dtype=jnp.float32)
    run_idx = jnp.zeros((1, _BT), dtype=jnp.int32)
    for c in range(_K // _CK):
        a = jax.lax.dot_general(
            cb_ref[c * _CK:(c + 1) * _CK, :], hnegt,
            (((1,), (1,)), ((), ())), preferred_element_type=jnp.float32)
        d2 = (h2 + a) + c2_ref[c * _CK:(c + 1) * _CK, :]
        cmin = jnp.min(d2, axis=0, keepdims=True)
        cidx = jnp.argmin(d2, axis=0).astype(jnp.int32).reshape(1, _BT) + c * _CK
        better = cmin < run_min
        run_idx = jnp.where(better, cidx, run_idx)
        run_min = jnp.where(better, cmin, run_min)
    idx = run_idx.reshape(_BT)
    out_ref[0, 0, :] = jnp.where(m_ref[0, 0, :] != 0, idx, -1)

    h = jnp.dot(p_ref[...], we_ref[...], preferred_element_type=jnp.float32) + be_ref[...]
    h = jax.nn.gelu(h)
    h = jnp.dot(h, w1_ref[...], preferred_element_type=jnp.float32) + b1_ref[...]
    h = jax.nn.gelu(h)
    h = jnp.dot(h, w2_ref[...], preferred_element_type=jnp.float32) + b2_ref[...]
    h2_ref[...] = jnp.sum(h * h, axis=1, keepdims=True).reshape(1, _BT)
    hnegt_ref[...] = -2.0 * h


def kernel(x, mask, W_e, b_e, W1, b1, W2, b2, codebook):
    xp = x.reshape(_B, _C, _H // _P, _P, _W // _P, _P)
    xp = xp.transpose(0, 2, 4, 1, 3, 5).reshape(_T, _PATCH_DIM)
    m = mask.reshape(_NB, 1, _BT).astype(jnp.int32)
    out = pl.pallas_call(
        _fused_kernel,
        grid=(_NB + 1,),
        in_specs=[
            pl.BlockSpec((_BT, _PATCH_DIM),
                         lambda i: (jnp.minimum(i, _NB - 1), 0)),
            pl.BlockSpec((_PATCH_DIM, _HID), lambda i: (0, 0)),
            pl.BlockSpec((1, _HID), lambda i: (0, 0)),
            pl.BlockSpec((_HID, _FF), lambda i: (0, 0)),
            pl.BlockSpec((1, _FF), lambda i: (0, 0)),
            pl.BlockSpec((_FF, _D), lambda i: (0, 0)),
            pl.BlockSpec((1, _D), lambda i: (0, 0)),
            pl.BlockSpec((_K, _D), lambda i: (0, 0)),
            pl.BlockSpec((1, 1, _BT), lambda i: (jnp.maximum(i - 1, 0), 0, 0)),
        ],
        out_specs=pl.BlockSpec((1, 1, _BT), lambda i: (jnp.maximum(i - 1, 0), 0, 0)),
        out_shape=jax.ShapeDtypeStruct((_NB, 1, _BT), jnp.int32),
        scratch_shapes=[
            pltpu.VMEM((_K, 1), jnp.float32),
            pltpu.VMEM((_BT, _D), jnp.float32),
            pltpu.VMEM((1, _BT), jnp.float32),
        ],
    )(xp, W_e, b_e.reshape(1, _HID), W1, b1.reshape(1, _FF), W2,
      b2.reshape(1, _D), codebook, m)
    return out.reshape(_B, _N_TOK)


# speedup vs baseline: 1.3857x; 1.3686x over previous
import jax
import jax.numpy as jnp
from jax.experimental import pallas as pl
from jax.experimental.pallas import tpu as pltpu

_B, _C, _H, _W = 16, 3, 384, 384
_P = 16
_PATCH_DIM = _C * _P * _P
_HID = 768
_FF = 1536
_D = 256
_K = 8192
_N_TOK = (_H // _P) * (_W // _P)
_T = _B * _N_TOK
_BT = 256
_NB = _T // _BT
_CK = 2048


def _fused_kernel(p_ref, we_ref, be_ref, w1_ref, b1_ref, w2_ref, b2_ref,
                  cb_ref, m_ref, out_ref, c2_ref, hnegt_ref, h2_ref):
    @pl.when(pl.program_id(0) == 0)
    def _():
        cb = cb_ref[...]
        c2_ref[...] = jnp.sum(cb * cb, axis=1, keepdims=True)

    hnegt = hnegt_ref[...]
    h2 = h2_ref[...]
    run_min = jnp.full((1, _BT), jnp.inf, dtype=jnp.float32)
    run_idx = jnp.zeros((1, _BT), dtype=jnp.int32)
    for c in range(_K // _CK):
        a = jax.lax.dot_general(
            cb_ref[c * _CK:(c + 1) * _CK, :], hnegt,
            (((1,), (1,)), ((), ())), preferred_element_type=jnp.float32)
        d2 = (h2 + a) + c2_ref[c * _CK:(c + 1) * _CK, :]
        cmin = jnp.min(d2, axis=0, keepdims=True)
        cidx = jnp.argmin(d2, axis=0).astype(jnp.int32).reshape(1, _BT) + c * _CK
        better = cmin < run_min
        run_idx = jnp.where(better, cidx, run_idx)
        run_min = jnp.where(better, cmin, run_min)
    idx = run_idx.reshape(_BT)
    out_ref[0, 0, :] = jnp.where(m_ref[0, 0, :] != 0, idx, -1)

    h = jnp.dot(p_ref[...], we_ref[...], preferred_element_type=jnp.float32) + be_ref[...]
    h = jax.nn.gelu(h)
    h = jnp.dot(h, w1_ref[...], preferred_element_type=jnp.float32) + b1_ref[...]
    h = jax.nn.gelu(h)
    h = jnp.dot(h, w2_ref[...], preferred_element_type=jnp.float32) + b2_ref[...]
    h2_ref[...] = jnp.sum(h * h, axis=1, keepdims=True).reshape(1, _BT)
    hnegt_ref[...] = -2.0 * h


def _patchify_kernel(x_ref, out_ref):
    v = x_ref[0, 0]
    out_ref[0, :, :, :] = v.transpose(0, 2, 1, 3).reshape(_H // _P, _W // _P,
                                                          _P * _P)


def _patchify(x):
    x6 = x.reshape(_B, _C, _H // _P, _P, _W // _P, _P)
    out = pl.pallas_call(
        _patchify_kernel,
        grid=(_B, _C),
        in_specs=[pl.BlockSpec((1, 1, _H // _P, _P, _W // _P, _P),
                               lambda b, c: (b, c, 0, 0, 0, 0))],
        out_specs=pl.BlockSpec((1, _H // _P, _W // _P, _P * _P),
                               lambda b, c: (b, 0, 0, c)),
        out_shape=jax.ShapeDtypeStruct(
            (_B, _H // _P, _W // _P, _C * _P * _P), jnp.float32),
    )(x6)
    return out.reshape(_T, _PATCH_DIM)


def kernel(x, mask, W_e, b_e, W1, b1, W2, b2, codebook):
    xp = _patchify(x)
    m = mask.reshape(_NB, 1, _BT).astype(jnp.int32)
    out = pl.pallas_call(
        _fused_kernel,
        grid=(_NB + 1,),
        in_specs=[
            pl.BlockSpec((_BT, _PATCH_DIM),
                         lambda i: (jnp.minimum(i, _NB - 1), 0)),
            pl.BlockSpec((_PATCH_DIM, _HID), lambda i: (0, 0)),
            pl.BlockSpec((1, _HID), lambda i: (0, 0)),
            pl.BlockSpec((_HID, _FF), lambda i: (0, 0)),
            pl.BlockSpec((1, _FF), lambda i: (0, 0)),
            pl.BlockSpec((_FF, _D), lambda i: (0, 0)),
            pl.BlockSpec((1, _D), lambda i: (0, 0)),
            pl.BlockSpec((_K, _D), lambda i: (0, 0)),
            pl.BlockSpec((1, 1, _BT), lambda i: (jnp.maximum(i - 1, 0), 0, 0)),
        ],
        out_specs=pl.BlockSpec((1, 1, _BT), lambda i: (jnp.maximum(i - 1, 0), 0, 0)),
        out_shape=jax.ShapeDtypeStruct((_NB, 1, _BT), jnp.int32),
        scratch_shapes=[
            pltpu.VMEM((_K, 1), jnp.float32),
            pltpu.VMEM((_BT, _D), jnp.float32),
            pltpu.VMEM((1, _BT), jnp.float32),
        ],
    )(xp, W_e, b_e.reshape(1, _HID), W1, b1.reshape(1, _FF), W2,
      b2.reshape(1, _D), codebook, m)
    return out.reshape(_B, _N_TOK)


# speedup vs baseline: 1.3981x; 1.0089x over previous
import jax
import jax.numpy as jnp
from jax.experimental import pallas as pl
from jax.experimental.pallas import tpu as pltpu

_B, _C, _H, _W = 16, 3, 384, 384
_P = 16
_PATCH_DIM = _C * _P * _P
_HID = 768
_FF = 1536
_D = 256
_K = 8192
_N_TOK = (_H // _P) * (_W // _P)
_T = _B * _N_TOK
_BT = _N_TOK
_NB = _B
_CK = 2048
_G = _H // _P


def _fused_kernel(x_ref, we_ref, be_ref, w1_ref, b1_ref, w2_ref, b2_ref,
                  cb_ref, m_ref, out_ref, c2_ref, p_ref, hneg_ref, h2_ref):
    @pl.when(pl.program_id(0) == 0)
    def _():
        cb = cb_ref[...]
        c2_ref[...] = jnp.sum(cb * cb, axis=1, keepdims=True)

    hneg = hneg_ref[...]
    h2 = h2_ref[...]
    run_min = jnp.full((1, _BT), jnp.inf, dtype=jnp.float32)
    run_idx = jnp.zeros((1, _BT), dtype=jnp.int32)
    for c in range(_K // _CK):
        a = jax.lax.dot_general(
            cb_ref[c * _CK:(c + 1) * _CK, :], hneg,
            (((1,), (1,)), ((), ())), preferred_element_type=jnp.float32)
        d2 = (h2 + a) + c2_ref[c * _CK:(c + 1) * _CK, :]
        cmin = jnp.min(d2, axis=0, keepdims=True)
        cidx = jnp.argmin(d2, axis=0).astype(jnp.int32).reshape(1, _BT) + c * _CK
        better = cmin < run_min
        run_idx = jnp.where(better, cidx, run_idx)
        run_min = jnp.where(better, cmin, run_min)
    idx = run_idx.reshape(_BT)
    out_ref[0, 0, :] = jnp.where(m_ref[0, 0, :] != 0, idx, -1)

    h = jnp.dot(p_ref[...], we_ref[...], preferred_element_type=jnp.float32) + be_ref[...]
    h = jax.nn.gelu(h)
    h = jnp.dot(h, w1_ref[...], preferred_element_type=jnp.float32) + b1_ref[...]
    h = jax.nn.gelu(h)
    h = jnp.dot(h, w2_ref[...], preferred_element_type=jnp.float32) + b2_ref[...]
    h2_ref[...] = jnp.sum(h * h, axis=1, keepdims=True).reshape(1, _BT)
    hneg_ref[...] = -2.0 * h

    v = x_ref[0]
    for c in range(_C):
        pc = v[c].transpose(0, 2, 1, 3).reshape(_BT, _P * _P)
        p_ref[:, c * _P * _P:(c + 1) * _P * _P] = pc


def kernel(x, mask, W_e, b_e, W1, b1, W2, b2, codebook):
    x6 = x.reshape(_B, _C, _G, _P, _G, _P)
    m = mask.reshape(_NB, 1, _BT).astype(jnp.int32)
    out = pl.pallas_call(
        _fused_kernel,
        grid=(_NB + 2,),
        in_specs=[
            pl.BlockSpec((1, _C, _G, _P, _G, _P),
                         lambda i: (jnp.minimum(i, _NB - 1), 0, 0, 0, 0, 0)),
            pl.BlockSpec((_PATCH_DIM, _HID), lambda i: (0, 0)),
            pl.BlockSpec((1, _HID), lambda i: (0, 0)),
            pl.BlockSpec((_HID, _FF), lambda i: (0, 0)),
            pl.BlockSpec((1, _FF), lambda i: (0, 0)),
            pl.BlockSpec((_FF, _D), lambda i: (0, 0)),
            pl.BlockSpec((1, _D), lambda i: (0, 0)),
            pl.BlockSpec((_K, _D), lambda i: (0, 0)),
            pl.BlockSpec((1, 1, _BT),
                         lambda i: (jnp.clip(i - 2, 0, _NB - 1), 0, 0)),
        ],
        out_specs=pl.BlockSpec((1, 1, _BT),
                               lambda i: (jnp.clip(i - 2, 0, _NB - 1), 0, 0)),
        out_shape=jax.ShapeDtypeStruct((_NB, 1, _BT), jnp.int32),
        scratch_shapes=[
            pltpu.VMEM((_K, 1), jnp.float32),
            pltpu.VMEM((_BT, _PATCH_DIM), jnp.float32),
            pltpu.VMEM((_BT, _D), jnp.float32),
            pltpu.VMEM((1, _BT), jnp.float32),
        ],
    )(x6, W_e, b_e.reshape(1, _HID), W1, b1.reshape(1, _FF), W2,
      b2.reshape(1, _D), codebook, m)
    return out.reshape(_B, _N_TOK)
